# baseline (speedup 1.0000x reference)
import functools
import math

import jax
import jax.numpy as jnp
from jax import lax
from jax.experimental import pallas as pl
from jax.experimental.pallas import tpu as pltpu
from jax.experimental.pallas import tpu_sc as plsc

_DM = 64
_SCALE = math.sqrt(_DM)
_BW = 128
_L = 16
_TCOLS = 2048


def _pair_transpose_tc(tt):
    dm, vocab = tt.shape
    grid = (vocab + _TCOLS - 1) // _TCOLS
    half = _TCOLS // 2

    def body(t_ref, o_ref):
        blk = t_ref[...]
        o_ref[:, 0:dm] = blk[:, 0:half].T
        o_ref[:, dm:2 * dm] = blk[:, half:_TCOLS].T

    return pl.pallas_call(
        body,
        grid=(grid,),
        in_specs=[pl.BlockSpec((dm, _TCOLS), lambda i: (0, i))],
        out_specs=pl.BlockSpec((half, 2 * dm), lambda i: (i, 0)),
        out_shape=jax.ShapeDtypeStruct((grid * half, 2 * dm), jnp.float32),
    )(tt)


_TG = 2


@functools.cache
def _make_lookup(n_t: int, n_b: int, nrows: int):
    info = plsc.get_sparse_core_info()
    nc, ns = info.num_cores, info.num_subcores
    nw = nc * ns
    assert n_b == nw * _BW and n_t % (2 * _TG) == 0
    n_steps = n_t // _TG

    mesh = plsc.VectorSubcoreMesh(core_axis_name="c", subcore_axis_name="s")

    @functools.partial(
        pl.kernel,
        mesh=mesh,
        out_type=jax.ShapeDtypeStruct((n_t, _DM // 8, nw, 8, _BW),
                                      jnp.float32),
        scratch_types=[
            pltpu.VMEM((n_t, _BW), jnp.int32),
            pltpu.VMEM((2, _TG, _BW), jnp.int32),
            pltpu.VMEM((2, _TG * _BW, _DM), jnp.float32),
            pltpu.VMEM((2, _TG, _DM // 8, 8, _BW), jnp.float32),
            pltpu.SemaphoreType.DMA,
            pltpu.SemaphoreType.DMA,
            pltpu.SemaphoreType.DMA,
            pltpu.SemaphoreType.DMA,
        ],
        compiler_params=pltpu.CompilerParams(
            use_tc_tiling_on_sc=False, needs_layout_passes=False),
    )
    def lookup(xt_hbm, tbl_hbm, out_hbm, xs_v, pv, rows_v, ob_v,
               gsem0, gsem1, osem0, osem1):
        wid = lax.axis_index("s") * nc + lax.axis_index("c")
        b0 = wid * _BW
        gsem = (gsem0, gsem1)
        osem = (osem0, osem1)

        def prep_idx(step, s):
            for tl in range(_TG):
                for k in range(_BW // _L):
                    sl = pl.ds(k * _L, _L)
                    v = xs_v[step * _TG + tl, sl]
                    pv[s, tl, sl] = (
                        lax.shift_left(lax.shift_right_logical(v, 11), 11)
                        + lax.shift_left(jnp.bitwise_and(v, 1023), 1)
                        + jnp.bitwise_and(lax.shift_right_logical(v, 10), 1))

        def gather_cps(s):
            half = _BW // 2
            return [
                pltpu.make_async_copy(
                    tbl_hbm.at[pv.at[s, tl, pl.ds(j * half, half)]],
                    rows_v.at[s, pl.ds(tl * _BW + j * half, half)], gsem[s])
                for tl in range(_TG) for j in range(2)
            ]

        def gather_start(s):
            for cp in gather_cps(s):
                cp.start()

        def gather_wait(s):
            for cp in gather_cps(s):
                cp.wait()

        def transpose_scale(s):
            iota = lax.iota(jnp.int32, _L)
            dt_idx = [lax.shift_right_logical(iota + k * _L, 3)
                      for k in range(_DM // _L)]
            d8_idx = [jnp.bitwise_and(iota + k * _L, 7)
                      for k in range(_DM // _L)]
            runroll = 4

            def r_body(r0, _):
                for tl in range(_TG):
                    for u in range(runroll):
                        b = r0 * runroll + u
                        bvec = jnp.full((_L,), 0, jnp.int32) + b
                        loads = [rows_v[s, tl * _BW + b, pl.ds(k * _L, _L)]
                                 for k in range(_DM // _L)]
                        for k in range(_DM // _L):
                            plsc.store_scatter(
                                ob_v.at[s, tl], [dt_idx[k], d8_idx[k], bvec],
                                loads[k] * _SCALE)
                return 0
            lax.fori_loop(0, _BW // runroll, r_body, 0)

        def out_cp(step, s):
            return pltpu.make_async_copy(
                ob_v.at[s], out_hbm.at[pl.ds(step * _TG, _TG), :, wid],
                osem[s])

        pltpu.sync_copy(xt_hbm.at[:, pl.ds(b0, _BW)], xs_v)

        prep_idx(0, 0)
        gather_start(0)
        prep_idx(1, 1)
        gather_start(1)
        gather_wait(0)
        transpose_scale(0)
        out_cp(0, 0).start()

        def pair_body(i, _):
            for s, off in ((0, 2), (1, 3)):
                step = 2 * i + off
                prep_idx(step, s)
                out_cp(step - 2, s).wait()
                gather_start(s)
                gather_wait(1 - s)
                transpose_scale(1 - s)
                out_cp(step - 1, 1 - s).start()
            return 0

        lax.fori_loop(0, (n_steps - 2) // 2, pair_body, 0)

        gather_wait(1)
        transpose_scale(1)
        out_cp(n_steps - 1, 1).start()
        out_cp(n_steps - 2, 0).wait()
        out_cp(n_steps - 1, 1).wait()

    return lookup


def kernel(x, table):
    n_b, n_t = x.shape
    vocab, dm = table.shape
    tbl2 = _pair_transpose_tc(table.T)
    tbl3 = tbl2.reshape(2 * tbl2.shape[0], dm)
    xt = x.T
    out5 = _make_lookup(n_t, n_b, tbl3.shape[0])(xt, tbl3)
    return out5.transpose(2, 4, 0, 1, 3).reshape(n_b, n_t, dm)


# speedup vs baseline: 1.2408x; 1.2408x over previous
import functools
import math

import jax
import jax.numpy as jnp
from jax import lax
from jax.experimental import pallas as pl
from jax.experimental.pallas import tpu as pltpu
from jax.experimental.pallas import tpu_sc as plsc

_DM = 64
_SCALE = math.sqrt(_DM)
_BW = 128
_L = 16
_TCOLS = 2048


def _pair_transpose_tc(tt):
    dm, vocab = tt.shape
    grid = (vocab + _TCOLS - 1) // _TCOLS
    half = _TCOLS // 2

    def body(t_ref, o_ref):
        blk = t_ref[...]
        o_ref[:, 0:dm] = blk[:, 0:half].T
        o_ref[:, dm:2 * dm] = blk[:, half:_TCOLS].T

    return pl.pallas_call(
        body,
        grid=(grid,),
        in_specs=[pl.BlockSpec((dm, _TCOLS), lambda i: (0, i))],
        out_specs=pl.BlockSpec((half, 2 * dm), lambda i: (i, 0)),
        out_shape=jax.ShapeDtypeStruct((grid * half, 2 * dm), jnp.float32),
    )(tt)


_TG = 2


@functools.cache
def _make_lookup(n_t: int, n_b: int, nrows: int):
    info = plsc.get_sparse_core_info()
    nc, ns = info.num_cores, info.num_subcores
    nw = nc * ns
    assert n_b == nw * _BW and n_t % (2 * _TG) == 0
    n_steps = n_t // _TG

    mesh = plsc.VectorSubcoreMesh(core_axis_name="c", subcore_axis_name="s")

    @functools.partial(
        pl.kernel,
        mesh=mesh,
        out_type=jax.ShapeDtypeStruct((n_t, _DM // 8, nw, 8, _BW),
                                      jnp.float32),
        scratch_types=[
            pltpu.VMEM((n_t, _BW), jnp.int32),
            pltpu.VMEM((2, _TG, _BW), jnp.int32),
            pltpu.VMEM((2, _TG * _BW, _DM), jnp.float32),
            pltpu.VMEM((2, _TG, _DM // 8, 8, _BW), jnp.float32),
            pltpu.SemaphoreType.DMA,
            pltpu.SemaphoreType.DMA,
            pltpu.SemaphoreType.DMA,
            pltpu.SemaphoreType.DMA,
        ],
        compiler_params=pltpu.CompilerParams(
            use_tc_tiling_on_sc=False, needs_layout_passes=False,
            disable_bounds_checks=True),
    )
    def lookup(xt_hbm, tbl_hbm, out_hbm, xs_v, pv, rows_v, ob_v,
               gsem0, gsem1, osem0, osem1):
        wid = lax.axis_index("s") * nc + lax.axis_index("c")
        b0 = wid * _BW
        gsem = (gsem0, gsem1)
        osem = (osem0, osem1)

        def prep_idx(step, s):
            for tl in range(_TG):
                for k in range(_BW // _L):
                    sl = pl.ds(k * _L, _L)
                    v = xs_v[step * _TG + tl, sl]
                    pv[s, tl, sl] = (
                        lax.shift_left(lax.shift_right_logical(v, 11), 11)
                        + lax.shift_left(jnp.bitwise_and(v, 1023), 1)
                        + jnp.bitwise_and(lax.shift_right_logical(v, 10), 1))

        def gather_cps(s):
            half = _BW // 2
            return [
                pltpu.make_async_copy(
                    tbl_hbm.at[pv.at[s, tl, pl.ds(j * half, half)]],
                    rows_v.at[s, pl.ds(tl * _BW + j * half, half)], gsem[s])
                for tl in range(_TG) for j in range(2)
            ]

        def gather_start(s):
            for cp in gather_cps(s):
                cp.start()

        def gather_wait(s):
            for cp in gather_cps(s):
                cp.wait()

        def transpose_scale(s):
            iota = lax.iota(jnp.int32, _L)
            cids = [jnp.full((_L,), d, jnp.int32) for d in range(_L)]

            def bg_body(bg, _):
                for tl in range(_TG):
                    rid = iota + (tl * _BW + bg * _L)
                    for dt in range(_DM // _L):
                        vals = [
                            plsc.load_gather(
                                rows_v.at[s], [rid, cids[dk] + dt * _L])
                            for dk in range(_L)
                        ]
                        vals = [v * _SCALE for v in vals]
                        for dk in range(_L):
                            d = dt * _L + dk
                            ob_v[s, tl, d // 8, d % 8,
                                 pl.ds(bg * _L, _L)] = vals[dk]
                return 0
            lax.fori_loop(0, _BW // _L, bg_body, 0)

        def out_cp(step, s):
            return pltpu.make_async_copy(
                ob_v.at[s], out_hbm.at[pl.ds(step * _TG, _TG), :, wid],
                osem[s])

        pltpu.sync_copy(xt_hbm.at[:, pl.ds(b0, _BW)], xs_v)

        prep_idx(0, 0)
        gather_start(0)
        prep_idx(1, 1)
        gather_start(1)
        gather_wait(0)
        transpose_scale(0)
        out_cp(0, 0).start()

        def pair_body(i, _):
            for s, off in ((0, 2), (1, 3)):
                step = 2 * i + off
                prep_idx(step, s)
                out_cp(step - 2, s).wait()
                gather_start(s)
                gather_wait(1 - s)
                transpose_scale(1 - s)
                out_cp(step - 1, 1 - s).start()
            return 0

        lax.fori_loop(0, (n_steps - 2) // 2, pair_body, 0)

        gather_wait(1)
        transpose_scale(1)
        out_cp(n_steps - 1, 1).start()
        out_cp(n_steps - 2, 0).wait()
        out_cp(n_steps - 1, 1).wait()

    return lookup


def kernel(x, table):
    n_b, n_t = x.shape
    vocab, dm = table.shape
    tbl2 = _pair_transpose_tc(table.T)
    tbl3 = tbl2.reshape(2 * tbl2.shape[0], dm)
    xt = x.T
    out5 = _make_lookup(n_t, n_b, tbl3.shape[0])(xt, tbl3)
    return out5.transpose(2, 4, 0, 1, 3).reshape(n_b, n_t, dm)


# speedup vs baseline: 1.8998x; 1.5311x over previous
import functools
import math

import jax
import jax.numpy as jnp
from jax import lax
from jax.experimental import pallas as pl
from jax.experimental.pallas import tpu as pltpu
from jax.experimental.pallas import tpu_sc as plsc

_DM = 64
_SCALE = math.sqrt(_DM)
_BW = 128
_L = 16
_TCOLS = 2048


def _pair_transpose_tc(tt):
    dm, vocab = tt.shape
    grid = (vocab + _TCOLS - 1) // _TCOLS
    half = _TCOLS // 2

    def body(t_ref, o_ref):
        blk = t_ref[...]
        o_ref[:, 0:dm] = blk[:, 0:half].T
        o_ref[:, dm:2 * dm] = blk[:, half:_TCOLS].T

    return pl.pallas_call(
        body,
        grid=(grid,),
        in_specs=[pl.BlockSpec((dm, _TCOLS), lambda i: (0, i))],
        out_specs=pl.BlockSpec((half, 2 * dm), lambda i: (i, 0)),
        out_shape=jax.ShapeDtypeStruct((grid * half, 2 * dm), jnp.float32),
    )(tt)


_TG = 2


@functools.cache
def _make_lookup(n_t: int, n_b: int, nrows: int):
    info = plsc.get_sparse_core_info()
    nc, ns = info.num_cores, info.num_subcores
    nw = nc * ns
    assert n_b == nw * _BW and n_t % (2 * _TG) == 0
    n_steps = n_t // _TG

    mesh = plsc.VectorSubcoreMesh(core_axis_name="c", subcore_axis_name="s")

    @functools.partial(
        pl.kernel,
        mesh=mesh,
        out_type=jax.ShapeDtypeStruct((n_t, _DM // 8, nw, 8, _BW),
                                      jnp.float32),
        scratch_types=[
            pltpu.VMEM((n_t, _BW), jnp.int32),
            pltpu.VMEM((2, _TG, _BW), jnp.int32),
            pltpu.VMEM((2, _TG * _BW, _DM), jnp.float32),
            pltpu.VMEM((2, _TG, _DM // 8, 8, _BW + 1), jnp.float32),
            pltpu.SemaphoreType.DMA,
            pltpu.SemaphoreType.DMA,
            pltpu.SemaphoreType.DMA,
            pltpu.SemaphoreType.DMA,
        ],
        compiler_params=pltpu.CompilerParams(
            use_tc_tiling_on_sc=False, needs_layout_passes=False,
            disable_bounds_checks=True),
    )
    def lookup(xt_hbm, tbl_hbm, out_hbm, xs_v, pv, rows_v, ob_v,
               gsem0, gsem1, osem0, osem1):
        wid = lax.axis_index("s") * nc + lax.axis_index("c")
        b0 = wid * _BW
        gsem = (gsem0, gsem1)
        osem = (osem0, osem1)

        def prep_idx(step, s):
            for tl in range(_TG):
                for k in range(_BW // _L):
                    sl = pl.ds(k * _L, _L)
                    v = xs_v[step * _TG + tl, sl]
                    pv[s, tl, sl] = (
                        lax.shift_left(lax.shift_right_logical(v, 11), 11)
                        + lax.shift_left(jnp.bitwise_and(v, 1023), 1)
                        + jnp.bitwise_and(lax.shift_right_logical(v, 10), 1))

        def gather_cps(s):
            half = _BW // 2
            return [
                pltpu.make_async_copy(
                    tbl_hbm.at[pv.at[s, tl, pl.ds(j * half, half)]],
                    rows_v.at[s, pl.ds(tl * _BW + j * half, half)], gsem[s])
                for tl in range(_TG) for j in range(2)
            ]

        def gather_start(s):
            for cp in gather_cps(s):
                cp.start()

        def gather_wait(s):
            for cp in gather_cps(s):
                cp.wait()

        def transpose_scale(s):
            iota = lax.iota(jnp.int32, _L)
            dt_idx = [lax.shift_right_logical(iota + k * _L, 3)
                      for k in range(_DM // _L)]
            d8_idx = [jnp.bitwise_and(iota + k * _L, 7)
                      for k in range(_DM // _L)]
            runroll = 4

            def r_body(r0, _):
                for tl in range(_TG):
                    for u in range(runroll):
                        b = r0 * runroll + u
                        bvec = iota * 0 + b
                        loads = [rows_v[s, tl * _BW + b, pl.ds(k * _L, _L)]
                                 for k in range(_DM // _L)]
                        for k in range(_DM // _L):
                            plsc.store_scatter(
                                ob_v.at[s, tl], [dt_idx[k], d8_idx[k], bvec],
                                loads[k] * _SCALE)
                return 0
            lax.fori_loop(0, _BW // runroll, r_body, 0)

        def out_cp(step, s):
            return pltpu.make_async_copy(
                ob_v.at[s, :, :, :, pl.ds(0, _BW)],
                out_hbm.at[pl.ds(step * _TG, _TG), :, wid],
                osem[s])

        pltpu.sync_copy(xt_hbm.at[:, pl.ds(b0, _BW)], xs_v)

        prep_idx(0, 0)
        gather_start(0)
        prep_idx(1, 1)
        gather_start(1)
        gather_wait(0)
        transpose_scale(0)
        out_cp(0, 0).start()

        def pair_body(i, _):
            for s, off in ((0, 2), (1, 3)):
                step = 2 * i + off
                prep_idx(step, s)
                out_cp(step - 2, s).wait()
                gather_start(s)
                gather_wait(1 - s)
                transpose_scale(1 - s)
                out_cp(step - 1, 1 - s).start()
            return 0

        lax.fori_loop(0, (n_steps - 2) // 2, pair_body, 0)

        gather_wait(1)
        transpose_scale(1)
        out_cp(n_steps - 1, 1).start()
        out_cp(n_steps - 2, 0).wait()
        out_cp(n_steps - 1, 1).wait()

    return lookup


def kernel(x, table):
    n_b, n_t = x.shape
    vocab, dm = table.shape
    tbl2 = _pair_transpose_tc(table.T)
    tbl3 = tbl2.reshape(2 * tbl2.shape[0], dm)
    xt = x.T
    out5 = _make_lookup(n_t, n_b, tbl3.shape[0])(xt, tbl3)
    return out5.transpose(2, 4, 0, 1, 3).reshape(n_b, n_t, dm)


# speedup vs baseline: 2.4654x; 1.2977x over previous
import functools
import math

import jax
import jax.numpy as jnp
from jax import lax
from jax.experimental import pallas as pl
from jax.experimental.pallas import tpu as pltpu
from jax.experimental.pallas import tpu_sc as plsc

_DM = 64
_SCALE = math.sqrt(_DM)
_BW = 128
_L = 16
_TCOLS = 8192
_SH = _TCOLS.bit_length() - 1
_HM = _TCOLS // 2 - 1


def _pair_transpose_tc(tt):
    dm, vocab = tt.shape
    grid = (vocab + _TCOLS - 1) // _TCOLS
    half = _TCOLS // 2

    def body(t_ref, o_ref):
        blk = t_ref[...]
        o_ref[:, 0:dm] = blk[:, 0:half].T
        o_ref[:, dm:2 * dm] = blk[:, half:_TCOLS].T

    return pl.pallas_call(
        body,
        grid=(grid,),
        in_specs=[pl.BlockSpec((dm, _TCOLS), lambda i: (0, i))],
        out_specs=pl.BlockSpec((half, 2 * dm), lambda i: (i, 0)),
        out_shape=jax.ShapeDtypeStruct((grid * half, 2 * dm), jnp.float32),
    )(tt)


_TG = 2


@functools.cache
def _make_lookup(n_t: int, n_b: int, nrows: int):
    info = plsc.get_sparse_core_info()
    nc, ns = info.num_cores, info.num_subcores
    nw = nc * ns
    assert n_b == nw * _BW and n_t % (2 * _TG) == 0
    n_steps = n_t // _TG

    mesh = plsc.VectorSubcoreMesh(core_axis_name="c", subcore_axis_name="s")

    @functools.partial(
        pl.kernel,
        mesh=mesh,
        out_type=jax.ShapeDtypeStruct((n_t, _DM // 8, nw, 8, _BW),
                                      jnp.float32),
        scratch_types=[
            pltpu.VMEM((n_t, _BW), jnp.int32),
            pltpu.VMEM((2, _TG, _BW), jnp.int32),
            pltpu.VMEM((2, _TG * _BW, _DM), jnp.float32),
            pltpu.VMEM((2, _TG, _DM // 8, 8, _BW + 1), jnp.float32),
            pltpu.SemaphoreType.DMA,
            pltpu.SemaphoreType.DMA,
            pltpu.SemaphoreType.DMA,
            pltpu.SemaphoreType.DMA,
        ],
        compiler_params=pltpu.CompilerParams(
            use_tc_tiling_on_sc=False, needs_layout_passes=False,
            disable_bounds_checks=True),
    )
    def lookup(xt_hbm, tbl_hbm, out_hbm, xs_v, pv, rows_v, ob_v,
               gsem0, gsem1, osem0, osem1):
        wid = lax.axis_index("s") * nc + lax.axis_index("c")
        b0 = wid * _BW
        gsem = (gsem0, gsem1)
        osem = (osem0, osem1)

        def prep_idx(step, s):
            for tl in range(_TG):
                for k in range(_BW // _L):
                    sl = pl.ds(k * _L, _L)
                    v = xs_v[step * _TG + tl, sl]
                    pv[s, tl, sl] = (
                        lax.shift_left(lax.shift_right_logical(v, _SH), _SH)
                        + lax.shift_left(jnp.bitwise_and(v, _HM), 1)
                        + jnp.bitwise_and(
                            lax.shift_right_logical(v, _SH - 1), 1))

        def gather_cps(s):
            half = _BW // 2
            return [
                pltpu.make_async_copy(
                    tbl_hbm.at[pv.at[s, tl, pl.ds(j * half, half)]],
                    rows_v.at[s, pl.ds(tl * _BW + j * half, half)], gsem[s])
                for tl in range(_TG) for j in range(2)
            ]

        def gather_start(s):
            for cp in gather_cps(s):
                cp.start()

        def gather_wait(s):
            for cp in gather_cps(s):
                cp.wait()

        def transpose_scale(s):
            iota = lax.iota(jnp.int32, _L)
            dt_idx = [lax.shift_right_logical(iota + k * _L, 3)
                      for k in range(_DM // _L)]
            d8_idx = [jnp.bitwise_and(iota + k * _L, 7)
                      for k in range(_DM // _L)]
            runroll = 4

            def r_body(r0, _):
                for tl in range(_TG):
                    for u in range(runroll):
                        b = r0 * runroll + u
                        bvec = iota * 0 + b
                        loads = [rows_v[s, tl * _BW + b, pl.ds(k * _L, _L)]
                                 for k in range(_DM // _L)]
                        for k in range(_DM // _L):
                            plsc.store_scatter(
                                ob_v.at[s, tl], [dt_idx[k], d8_idx[k], bvec],
                                loads[k] * _SCALE)
                return 0
            lax.fori_loop(0, _BW // runroll, r_body, 0)

        def out_cp(step, s):
            return pltpu.make_async_copy(
                ob_v.at[s, :, :, :, pl.ds(0, _BW)],
                out_hbm.at[pl.ds(step * _TG, _TG), :, wid],
                osem[s])

        pltpu.sync_copy(xt_hbm.at[:, pl.ds(b0, _BW)], xs_v)

        prep_idx(0, 0)
        gather_start(0)
        prep_idx(1, 1)
        gather_start(1)
        gather_wait(0)
        transpose_scale(0)
        out_cp(0, 0).start()

        def pair_body(i, _):
            for s, off in ((0, 2), (1, 3)):
                step = 2 * i + off
                prep_idx(step, s)
                out_cp(step - 2, s).wait()
                gather_start(s)
                gather_wait(1 - s)
                transpose_scale(1 - s)
                out_cp(step - 1, 1 - s).start()
            return 0

        lax.fori_loop(0, (n_steps - 2) // 2, pair_body, 0)

        gather_wait(1)
        transpose_scale(1)
        out_cp(n_steps - 1, 1).start()
        out_cp(n_steps - 2, 0).wait()
        out_cp(n_steps - 1, 1).wait()

    return lookup


def kernel(x, table):
    n_b, n_t = x.shape
    vocab, dm = table.shape
    tbl2 = _pair_transpose_tc(table.T)
    tbl3 = tbl2.reshape(2 * tbl2.shape[0], dm)
    xt = x.T
    out5 = _make_lookup(n_t, n_b, tbl3.shape[0])(xt, tbl3)
    return out5.transpose(2, 4, 0, 1, 3).reshape(n_b, n_t, dm)


# speedup vs baseline: 2.6019x; 1.0554x over previous
import functools
import math

import jax
import jax.numpy as jnp
from jax import lax
from jax.experimental import pallas as pl
from jax.experimental.pallas import tpu as pltpu
from jax.experimental.pallas import tpu_sc as plsc

_DM = 64
_SCALE = math.sqrt(_DM)
_BW = 128
_L = 16
_TCOLS = 16384
_SH = _TCOLS.bit_length() - 1
_HM = _TCOLS // 2 - 1


def _pair_transpose_tc(tt):
    dm, vocab = tt.shape
    grid = (vocab + _TCOLS - 1) // _TCOLS
    half = _TCOLS // 2

    def body(t_ref, o_ref):
        blk = t_ref[...]
        o_ref[:, 0:dm] = blk[:, 0:half].T
        o_ref[:, dm:2 * dm] = blk[:, half:_TCOLS].T

    return pl.pallas_call(
        body,
        grid=(grid,),
        in_specs=[pl.BlockSpec((dm, _TCOLS), lambda i: (0, i))],
        out_specs=pl.BlockSpec((half, 2 * dm), lambda i: (i, 0)),
        out_shape=jax.ShapeDtypeStruct((grid * half, 2 * dm), jnp.float32),
    )(tt)


_TG = 2


@functools.cache
def _make_lookup(n_t: int, n_b: int, nrows: int):
    info = plsc.get_sparse_core_info()
    nc, ns = info.num_cores, info.num_subcores
    nw = nc * ns
    assert n_b == nw * _BW and n_t % (2 * _TG) == 0
    n_steps = n_t // _TG

    mesh = plsc.VectorSubcoreMesh(core_axis_name="c", subcore_axis_name="s")

    @functools.partial(
        pl.kernel,
        mesh=mesh,
        out_type=jax.ShapeDtypeStruct((n_t, _DM // 8, nw, 8, _BW),
                                      jnp.float32),
        scratch_types=[
            pltpu.VMEM((n_t, _BW), jnp.int32),
            pltpu.VMEM((2, _TG, _BW), jnp.int32),
            pltpu.VMEM((2, _TG * _BW, _DM), jnp.float32),
            pltpu.VMEM((2, _TG, _DM // 8, 8, _BW + 1), jnp.float32),
            pltpu.SemaphoreType.DMA,
            pltpu.SemaphoreType.DMA,
            pltpu.SemaphoreType.DMA,
            pltpu.SemaphoreType.DMA,
        ],
        compiler_params=pltpu.CompilerParams(
            use_tc_tiling_on_sc=False, needs_layout_passes=False,
            disable_bounds_checks=True),
    )
    def lookup(xt_hbm, tbl_hbm, out_hbm, xs_v, pv, rows_v, ob_v,
               gsem0, gsem1, osem0, osem1):
        wid = lax.axis_index("s") * nc + lax.axis_index("c")
        b0 = wid * _BW
        gsem = (gsem0, gsem1)
        osem = (osem0, osem1)

        def prep_idx(step, s):
            for tl in range(_TG):
                for k in range(_BW // _L):
                    sl = pl.ds(k * _L, _L)
                    v = xs_v[step * _TG + tl, sl]
                    pv[s, tl, sl] = (
                        lax.shift_left(lax.shift_right_logical(v, _SH), _SH)
                        + lax.shift_left(jnp.bitwise_and(v, _HM), 1)
                        + jnp.bitwise_and(
                            lax.shift_right_logical(v, _SH - 1), 1))

        def gather_cps(s):
            half = _BW // 2
            return [
                pltpu.make_async_copy(
                    tbl_hbm.at[pv.at[s, tl, pl.ds(j * half, half)]],
                    rows_v.at[s, pl.ds(tl * _BW + j * half, half)], gsem[s])
                for tl in range(_TG) for j in range(2)
            ]

        def gather_start(s):
            for cp in gather_cps(s):
                cp.start()

        def gather_wait(s):
            for cp in gather_cps(s):
                cp.wait()

        def transpose_scale(s):
            iota = lax.iota(jnp.int32, _L)
            dt_idx = [lax.shift_right_logical(iota + k * _L, 3)
                      for k in range(_DM // _L)]
            d8_idx = [jnp.bitwise_and(iota + k * _L, 7)
                      for k in range(_DM // _L)]
            runroll = 4

            def r_body(r0, _):
                for tl in range(_TG):
                    for u in range(runroll):
                        b = r0 * runroll + u
                        bvec = iota * 0 + b
                        loads = [rows_v[s, tl * _BW + b, pl.ds(k * _L, _L)]
                                 for k in range(_DM // _L)]
                        for k in range(_DM // _L):
                            plsc.store_scatter(
                                ob_v.at[s, tl], [dt_idx[k], d8_idx[k], bvec],
                                loads[k] * _SCALE)
                return 0
            lax.fori_loop(0, _BW // runroll, r_body, 0)

        def out_cp(step, s):
            return pltpu.make_async_copy(
                ob_v.at[s, :, :, :, pl.ds(0, _BW)],
                out_hbm.at[pl.ds(step * _TG, _TG), :, wid],
                osem[s])

        pltpu.sync_copy(xt_hbm.at[:, pl.ds(b0, _BW)], xs_v)

        prep_idx(0, 0)
        gather_start(0)
        prep_idx(1, 1)
        gather_start(1)
        gather_wait(0)
        transpose_scale(0)
        out_cp(0, 0).start()

        def pair_body(i, _):
            for s, off in ((0, 2), (1, 3)):
                step = 2 * i + off
                prep_idx(step, s)
                out_cp(step - 2, s).wait()
                gather_start(s)
                gather_wait(1 - s)
                transpose_scale(1 - s)
                out_cp(step - 1, 1 - s).start()
            return 0

        lax.fori_loop(0, (n_steps - 2) // 2, pair_body, 0)

        gather_wait(1)
        transpose_scale(1)
        out_cp(n_steps - 1, 1).start()
        out_cp(n_steps - 2, 0).wait()
        out_cp(n_steps - 1, 1).wait()

    return lookup


def kernel(x, table):
    n_b, n_t = x.shape
    vocab, dm = table.shape
    tbl2 = _pair_transpose_tc(table.T)
    tbl3 = tbl2.reshape(2 * tbl2.shape[0], dm)
    xt = x.T
    out5 = _make_lookup(n_t, n_b, tbl3.shape[0])(xt, tbl3)
    return out5.transpose(2, 4, 0, 1, 3).reshape(n_b, n_t, dm)
